# baseline (speedup 1.0000x reference)
import jax
import jax.numpy as jnp
from jax.experimental import pallas as pl
from jax.experimental.pallas import tpu as pltpu

MAX_CTX = 32768
DIM = 2048

_N = MAX_CTX * DIM
_HALF = _N // 2
_NCHUNK = 8
_CSZ = _HALF // _NCHUNK


def _dma_body(mem_ref, h_ref, out_ref, sems):
    copies = []
    for k in range(_NCHUNK):
        copies.append(pltpu.make_async_copy(
            mem_ref.at[pl.ds(_HALF + k * _CSZ, _CSZ)],
            out_ref.at[pl.ds(k * _CSZ, _CSZ)],
            sems.at[2 * k]))
        copies.append(pltpu.make_async_copy(
            h_ref.at[pl.ds(k * _CSZ, _CSZ)],
            out_ref.at[pl.ds(_HALF + k * _CSZ, _CSZ)],
            sems.at[2 * k + 1]))
    for c in copies:
        c.start()
    for c in copies:
        c.wait()


def kernel(h, mem):
    B, L, D = h.shape
    flat = h.reshape(B * L * D)
    mem1 = mem.reshape(_N)
    new_mem = pl.pallas_call(
        _dma_body,
        in_specs=[
            pl.BlockSpec(memory_space=pltpu.MemorySpace.HBM),
            pl.BlockSpec(memory_space=pltpu.MemorySpace.HBM),
        ],
        out_specs=pl.BlockSpec(memory_space=pltpu.MemorySpace.HBM),
        out_shape=jax.ShapeDtypeStruct((_N,), h.dtype),
        scratch_shapes=[pltpu.SemaphoreType.DMA((2 * _NCHUNK,))],
    )(mem1, flat)
    return h, new_mem.reshape(MAX_CTX, DIM)


# speedup vs baseline: 31.8584x; 31.8584x over previous
import functools

import jax
import jax.numpy as jnp
from jax import lax
from jax.experimental import pallas as pl
from jax.experimental.pallas import tpu as pltpu
from jax.experimental.pallas import tpu_sc as plsc

MAX_CTX = 32768
DIM = 2048

_HALF_ROWS = MAX_CTX // 2

_TC_ROWS = 24576
_TC_BR = 1024
_TC_NBLK = _TC_ROWS // _TC_BR
_TC_MEMBLK = _HALF_ROWS // _TC_BR

_SC_ROWS = MAX_CTX - _TC_ROWS
_SC_WORKERS = 32
_SC_PER_W = _SC_ROWS // _SC_WORKERS
_SC_CH = 16
_SC_NCH = _SC_PER_W // _SC_CH

_mesh = plsc.VectorSubcoreMesh(core_axis_name="c", subcore_axis_name="s")


@functools.partial(
    pl.kernel,
    out_type=jax.ShapeDtypeStruct((MAX_CTX, DIM), jnp.float32),
    mesh=_mesh,
    scratch_types=[
        pltpu.VMEM((2, _SC_CH, DIM), jnp.float32),
        pltpu.SemaphoreType.DMA((2,)),
        pltpu.SemaphoreType.DMA((2,)),
    ],
)
def _sc_copy(h_hbm, out_hbm, buf, rsem, wsem):
    wid = lax.axis_index("c") * 16 + lax.axis_index("s")
    base = _TC_ROWS + wid * _SC_PER_W
    src_base = base - _HALF_ROWS

    def read(c, slot):
        return pltpu.make_async_copy(
            h_hbm.at[pl.ds(src_base + c * _SC_CH, _SC_CH), :],
            buf.at[slot], rsem.at[slot])

    def write(c, slot):
        return pltpu.make_async_copy(
            buf.at[slot],
            out_hbm.at[pl.ds(base + c * _SC_CH, _SC_CH), :], wsem.at[slot])

    read(0, 0).start()

    def step(c, _):
        slot = lax.rem(c, 2)
        nslot = lax.rem(c + 1, 2)
        read(c, slot).wait()

        @pl.when(c >= 1)
        def _():
            write(c - 1, nslot).wait()

        @pl.when(c + 1 < _SC_NCH)
        def _():
            read(c + 1, nslot).start()

        write(c, slot).start()
        return 0

    lax.fori_loop(0, _SC_NCH, step, 0)
    write(_SC_NCH - 1, (_SC_NCH - 1) % 2).wait()


def _tc_body(mem_ref, h_ref, sc_ref, out_ref):
    del sc_ref
    i = pl.program_id(0)

    @pl.when(i < _TC_MEMBLK)
    def _():
        out_ref[...] = mem_ref[...]

    @pl.when(i >= _TC_MEMBLK)
    def _():
        out_ref[...] = h_ref[...]


def kernel(h, mem):
    B, L, D = h.shape
    flat = h.reshape(B * L, D)
    sc_big = _sc_copy(flat)
    new_mem = pl.pallas_call(
        _tc_body,
        grid=(_TC_NBLK,),
        in_specs=[
            pl.BlockSpec((_TC_BR, D),
                         lambda i: (jnp.where(i < _TC_MEMBLK, i + _TC_MEMBLK,
                                              2 * _TC_MEMBLK - 1), 0)),
            pl.BlockSpec((_TC_BR, D),
                         lambda i: (jnp.where(i < _TC_MEMBLK, 0,
                                              i - _TC_MEMBLK), 0)),
            pl.BlockSpec(memory_space=pltpu.MemorySpace.HBM),
        ],
        out_specs=pl.BlockSpec((_TC_BR, D), lambda i: (i, 0)),
        out_shape=jax.ShapeDtypeStruct((MAX_CTX, D), h.dtype),
        input_output_aliases={2: 0},
    )(mem, flat, sc_big)
    return h, new_mem


# speedup vs baseline: 32.4229x; 1.0177x over previous
import functools

import jax
import jax.numpy as jnp
from jax import lax
from jax.experimental import pallas as pl
from jax.experimental.pallas import tpu as pltpu
from jax.experimental.pallas import tpu_sc as plsc

MAX_CTX = 32768
DIM = 2048

_HALF_ROWS = MAX_CTX // 2
_WORKERS = 32
_PER_W = MAX_CTX // _WORKERS
_CH = 32
_NCH = _PER_W // _CH

_mesh = plsc.VectorSubcoreMesh(core_axis_name="c", subcore_axis_name="s")


@functools.partial(
    pl.kernel,
    out_type=jax.ShapeDtypeStruct((MAX_CTX, DIM), jnp.float32),
    mesh=_mesh,
    scratch_types=[
        pltpu.VMEM_SHARED((16, 2, _CH, DIM), jnp.float32),
        pltpu.SemaphoreType.DMA((2,)),
        pltpu.SemaphoreType.DMA((2,)),
    ],
)
def _sc_copy(mem_hbm, h_hbm, out_hbm, buf, rsem, wsem):
    cid = lax.axis_index("c")
    sid = lax.axis_index("s")
    wid = cid * 16 + sid
    base = wid * _PER_W

    def copy_stripe(src_ref, src_base):
        def read(c, slot):
            return pltpu.make_async_copy(
                src_ref.at[pl.ds(src_base + c * _CH, _CH), :],
                buf.at[sid, slot], rsem.at[slot])

        def write(c, slot):
            return pltpu.make_async_copy(
                buf.at[sid, slot],
                out_hbm.at[pl.ds(base + c * _CH, _CH), :], wsem.at[slot])

        read(0, 0).start()

        def step(c, _):
            slot = lax.rem(c, 2)
            nslot = lax.rem(c + 1, 2)
            read(c, slot).wait()

            @pl.when(c >= 1)
            def _():
                write(c - 1, nslot).wait()

            @pl.when(c + 1 < _NCH)
            def _():
                read(c + 1, nslot).start()

            write(c, slot).start()
            return 0

        lax.fori_loop(0, _NCH, step, 0)
        write(_NCH - 1, (_NCH - 1) % 2).wait()

    @pl.when(wid < _WORKERS // 2)
    def _():
        copy_stripe(mem_hbm, base + _HALF_ROWS)

    @pl.when(wid >= _WORKERS // 2)
    def _():
        copy_stripe(h_hbm, base - _HALF_ROWS)


def kernel(h, mem):
    B, L, D = h.shape
    flat = h.reshape(B * L, D)
    new_mem = _sc_copy(mem, flat)
    return h, new_mem
